# baseline (speedup 1.0000x reference)
import functools

import jax
import jax.numpy as jnp
from jax.experimental import pallas as pl


def _gate_body(x_ref, w_ref, o_ref):
    x = x_ref[...].astype(jnp.bfloat16)
    w = w_ref[...].astype(jnp.bfloat16)
    acc = jax.lax.dot_general(
        x, w, (((1,), (1,)), ((), ())),
        preferred_element_type=jnp.float32)

    @pl.when(pl.program_id(1) == 0)
    def _():
        o_ref[...] = acc

    @pl.when(pl.program_id(1) == 1)
    def _():
        o_ref[...] += acc


@functools.partial(jax.jit, static_argnames=())
def kernel(x, W):
    tokens, hidden = x.shape
    experts = W.shape[0]
    bm = 1024
    kc = hidden // 2
    return pl.pallas_call(
        _gate_body,
        grid=(tokens // bm, 2),
        in_specs=[
            pl.BlockSpec((bm, kc), lambda i, k: (i, k)),
            pl.BlockSpec((experts, kc), lambda i, k: (0, k)),
        ],
        out_specs=pl.BlockSpec((bm, experts), lambda i, k: (i, 0)),
        out_shape=jax.ShapeDtypeStruct((tokens, experts), jnp.float32),
    )(x, W)


# speedup vs baseline: 1.1488x; 1.1488x over previous
import functools

import jax
import jax.numpy as jnp
from jax.experimental import pallas as pl


def _gate_body(x_ref, w_ref, o_ref):
    x = x_ref[...].astype(jnp.bfloat16)
    w = w_ref[...].astype(jnp.bfloat16)
    o_ref[...] = jax.lax.dot_general(
        x, w, (((1,), (1,)), ((), ())),
        preferred_element_type=jnp.float32)


@functools.partial(jax.jit, static_argnames=())
def kernel(x, W):
    tokens, hidden = x.shape
    experts = W.shape[0]
    bm = 1024
    return pl.pallas_call(
        _gate_body,
        grid=(tokens // bm,),
        in_specs=[
            pl.BlockSpec((bm, hidden), lambda i: (i, 0)),
            pl.BlockSpec((experts, hidden), lambda i: (0, 0)),
        ],
        out_specs=pl.BlockSpec((bm, experts), lambda i: (i, 0)),
        out_shape=jax.ShapeDtypeStruct((tokens, experts), jnp.float32),
    )(x, W)
